# Initial kernel scaffold; baseline (speedup 1.0000x reference)
import jax
import jax.numpy as jnp
from jax.experimental import pallas as pl


def kernel(x):
    raise NotImplementedError("write your pallas kernel here")


# speedup vs baseline: 13.1758x; 13.1758x over previous
import functools

import jax
import jax.numpy as jnp
from jax import lax
from jax.experimental import pallas as pl
from jax.experimental.pallas import tpu as pltpu
from jax.experimental.pallas import tpu_sc as plsc

B, T = 16384, 200
NC, NS, L = 2, 16, 16
NW = NC * NS
RPW = B // NW
GROUPS = RPW // L


@functools.partial(
    pl.kernel,
    out_type=jax.ShapeDtypeStruct((B, T), jnp.float32),
    mesh=plsc.VectorSubcoreMesh(core_axis_name="c", subcore_axis_name="s"),
    scratch_types=[pltpu.VMEM((RPW, T), jnp.float32)],
    compiler_params=pltpu.CompilerParams(
        use_tc_tiling_on_sc=False, needs_layout_passes=False
    ),
)
def _swap01(x_hbm, y_hbm, buf):
    wid = lax.axis_index("s") * NC + lax.axis_index("c")
    base = wid * RPW
    pltpu.sync_copy(x_hbm.at[pl.ds(base, RPW)], buf)
    lanes = lax.iota(jnp.int32, L)
    col0 = jnp.zeros((L,), jnp.int32)
    col1 = col0 + 1
    for g in range(GROUPS):
        rows = lanes + (g * L)
        v0 = plsc.load_gather(buf, [rows, col0])
        v1 = plsc.load_gather(buf, [rows, col1])
        plsc.store_scatter(buf, [rows, col0], v1)
        plsc.store_scatter(buf, [rows, col1], v0)
    pltpu.sync_copy(buf, y_hbm.at[pl.ds(base, RPW)])


def kernel(x):
    y = _swap01(x)
    return (x, y, x, y, x)
